# Initial kernel scaffold; baseline (speedup 1.0000x reference)
import jax
import jax.numpy as jnp
from jax.experimental import pallas as pl


def kernel(feats_node, feats_edge, edge_index, graph_node_index, graph_edge_index, feats_rho, curve_c_shape, curve_c_magnitude, W_in_n, b_in_n, W_in_e, b_in_e, W_msg, b_msg, W_upd, b_upd, W_eupd, b_eupd, W_r1, b_r1, W_r2, b_r2, W_s1, b_s1, W_s2, b_s2, W_m1, b_m1, W_m2, b_m2):
    raise NotImplementedError("write your pallas kernel here")


# speedup vs baseline: 2.7523x; 2.7523x over previous
import functools

import jax
import jax.numpy as jnp
from jax import lax
from jax.experimental import pallas as pl
from jax.experimental.pallas import tpu as pltpu
from jax.experimental.pallas import tpu_sc as plsc

N = 10000
E = 320000
B = 8
DIN = 128
DE = 16
D = 128
L = 3
R = 1024

BN = 1000
BE = 2560
NC = 2
NS = 16
NW = NC * NS
EW = E // NW
CH = 80
NCHUNK = EW // CH
NROWT = N // NS


def _elu(x):
    return jnp.where(x > 0, x, jnp.exp(x) - 1.0)


def _node_embed_body(x_ref, win_ref, bin_ref, w1_ref, w2_ref,
                     h_ref, p1_ref, p2_ref):
    h = _elu(jnp.dot(x_ref[...], win_ref[...],
                     preferred_element_type=jnp.float32) + bin_ref[...])
    h_ref[...] = h
    p1_ref[...] = jnp.dot(h, w1_ref[...], preferred_element_type=jnp.float32)
    p2_ref[...] = jnp.dot(h, w2_ref[...], preferred_element_type=jnp.float32)


def _node_embed(x, win, bin_, w1, w2, interpret=False):
    full = lambda s: pl.BlockSpec(s, lambda i: (0, 0))
    return pl.pallas_call(
        _node_embed_body,
        grid=(N // BN,),
        in_specs=[pl.BlockSpec((BN, DIN), lambda i: (i, 0)),
                  full((DIN, D)), full((1, D)), full((D, D)), full((D, D))],
        out_specs=[pl.BlockSpec((BN, D), lambda i: (i, 0))] * 3,
        out_shape=[jax.ShapeDtypeStruct((N, D), jnp.float32)] * 3,
        interpret=interpret,
    )(x, win, bin_, w1, w2)


def _edge_embed_body(x_ref, win_ref, bin_ref, w3_ref, bm_ref, e_ref, p3_ref):
    e = _elu(jnp.dot(x_ref[...], win_ref[...],
                     preferred_element_type=jnp.float32) + bin_ref[...])
    e_ref[...] = e
    p3_ref[...] = jnp.dot(e, w3_ref[...],
                          preferred_element_type=jnp.float32) + bm_ref[...]


def _edge_embed(x, win, bin_, w3, bm, interpret=False):
    full = lambda s: pl.BlockSpec(s, lambda i: (0, 0))
    return pl.pallas_call(
        _edge_embed_body,
        grid=(E // BE,),
        in_specs=[pl.BlockSpec((BE, DE), lambda i: (i, 0)),
                  full((DE, D)), full((1, D)), full((D, D)), full((1, D))],
        out_specs=[pl.BlockSpec((BE, D), lambda i: (i, 0))] * 2,
        out_shape=[jax.ShapeDtypeStruct((E, D), jnp.float32)] * 2,
        interpret=interpret,
    )(x, win, bin_, w3, bm)


def _update_body(h_ref, s0_ref, s1_ref, d0_ref, d1_ref,
                 wu1_ref, wu2_ref, bu_ref, w1_ref, w2_ref,
                 h_out, p1_out, p2_out):
    deg = jnp.maximum(d0_ref[:, :1] + d1_ref[:, :1], 1.0)
    agg = (s0_ref[...] + s1_ref[...]) / deg
    h = _elu(jnp.dot(h_ref[...], wu1_ref[...], preferred_element_type=jnp.float32)
             + jnp.dot(agg, wu2_ref[...], preferred_element_type=jnp.float32)
             + bu_ref[...])
    h_out[...] = h
    if p1_out is not None:
        p1_out[...] = jnp.dot(h, w1_ref[...], preferred_element_type=jnp.float32)
        p2_out[...] = jnp.dot(h, w2_ref[...], preferred_element_type=jnp.float32)


def _update(h, s0, s1, d0, d1, wu1, wu2, bu, w1, w2, last, interpret=False):
    full = lambda s: pl.BlockSpec(s, lambda i: (0, 0))
    nout = 1 if last else 3
    body = _update_body if not last else (
        lambda a, b, c, d, e, f, g, hh, i, j, k:
        _update_body(a, b, c, d, e, f, g, hh, i, j, k, None, None))
    out = pl.pallas_call(
        body,
        grid=(N // BN,),
        in_specs=[pl.BlockSpec((BN, D), lambda i: (i, 0))] * 3
        + [pl.BlockSpec((BN, 128), lambda i: (i, 0))] * 2
        + [full((D, D)), full((D, D)), full((1, D)), full((D, D)), full((D, D))],
        out_specs=[pl.BlockSpec((BN, D), lambda i: (i, 0))] * nout,
        out_shape=[jax.ShapeDtypeStruct((N, D), jnp.float32)] * nout,
        interpret=interpret,
    )(h, s0, s1, d0, d1, wu1, wu2, bu, w1, w2)
    return (out[0], None, None) if last else (out[0], out[1], out[2])


def _eupd_body(e_ref, m_ref, we1_ref, we2_ref, be_ref, w3_ref, bm_ref,
               e_out, p3_out):
    e = _elu(jnp.dot(e_ref[...], we1_ref[...], preferred_element_type=jnp.float32)
             + jnp.dot(m_ref[...], we2_ref[...], preferred_element_type=jnp.float32)
             + be_ref[...])
    e_out[...] = e
    if p3_out is not None:
        p3_out[...] = jnp.dot(e, w3_ref[...],
                              preferred_element_type=jnp.float32) + bm_ref[...]


def _eupd(e, m, we1, we2, be, w3, bm, last, interpret=False):
    full = lambda s: pl.BlockSpec(s, lambda i: (0, 0))
    nout = 1 if last else 2
    body = _eupd_body if not last else (
        lambda a, b, c, d, ee, f, g, hh: _eupd_body(a, b, c, d, ee, f, g, hh, None))
    out = pl.pallas_call(
        body,
        grid=(E // BE,),
        in_specs=[pl.BlockSpec((BE, D), lambda i: (i, 0))] * 2
        + [full((D, D)), full((D, D)), full((1, D)), full((D, D)), full((1, D))],
        out_specs=[pl.BlockSpec((BE, D), lambda i: (i, 0))] * nout,
        out_shape=[jax.ShapeDtypeStruct((E, D), jnp.float32)] * nout,
        interpret=interpret,
    )(e, m, we1, we2, be, w3, bm)
    return (out[0], None) if last else (out[0], out[1])


def _pool_body(gid_ref, x_ref, sum_ref, cnt_ref):
    i = pl.program_id(0)
    gid = gid_ref[0, :, :]
    rows = gid_ref.shape[2]
    onehot = (lax.broadcasted_iota(jnp.int32, (B, rows), 0)
              == gid).astype(jnp.float32)
    psum = jnp.dot(onehot, x_ref[...], preferred_element_type=jnp.float32)
    pcnt = jnp.sum(onehot, axis=1, keepdims=True)

    @pl.when(i == 0)
    def _():
        sum_ref[...] = jnp.zeros_like(sum_ref)
        cnt_ref[...] = jnp.zeros_like(cnt_ref)

    sum_ref[...] += psum
    cnt_ref[...] += jnp.broadcast_to(pcnt, cnt_ref.shape)


def _pool(gid3, x, blk, interpret=False):
    nblk = x.shape[0] // blk
    return pl.pallas_call(
        _pool_body,
        grid=(nblk,),
        in_specs=[pl.BlockSpec((1, 1, blk), lambda i: (i, 0, 0)),
                  pl.BlockSpec((blk, D), lambda i: (i, 0))],
        out_specs=[pl.BlockSpec((B, D), lambda i: (0, 0)),
                   pl.BlockSpec((B, 128), lambda i: (0, 0))],
        out_shape=[jax.ShapeDtypeStruct((B, D), jnp.float32),
                   jax.ShapeDtypeStruct((B, 128), jnp.float32)],
        interpret=interpret,
    )(gid3, x)


def _head_body(nsum_ref, ncnt_ref, esum_ref, ecnt_ref, rho_ref,
               wr1_ref, br1_ref, wr2_ref, br2_ref,
               ws1_ref, bs1_ref, ws2_ref, bs2_ref,
               wm1_ref, bm1_ref, wm2_ref, bm2_ref,
               tgt_s_ref, tgt_m_ref, out_ref):
    pn = nsum_ref[...] / jnp.maximum(ncnt_ref[:, :1], 1.0)
    pe = esum_ref[...] / jnp.maximum(ecnt_ref[:, :1], 1.0)
    emb = jnp.dot(_elu(rho_ref[...] * wr1_ref[...] + br1_ref[...]),
                  wr2_ref[...], preferred_element_type=jnp.float32) + br2_ref[...]
    x = pn + pe + emb
    cs = jnp.dot(_elu(jnp.dot(x, ws1_ref[...], preferred_element_type=jnp.float32)
                      + bs1_ref[...]),
                 ws2_ref[...], preferred_element_type=jnp.float32) + bs2_ref[...]
    cm = jnp.dot(_elu(jnp.dot(x, wm1_ref[...], preferred_element_type=jnp.float32)
                      + bm1_ref[...]),
                 wm2_ref[...], preferred_element_type=jnp.float32) + bm2_ref[...]
    loss_shape = jnp.sum((cs - tgt_s_ref[...]) ** 2) / (B * R)
    loss_mag = jnp.sum((cm - tgt_m_ref[...]) ** 2) / (B * 2)
    d = cs[:, :-1] - cs[:, 1:]
    loss_smooth = jnp.sum(d * d) / (B * (R - 1))
    total = loss_mag + loss_shape + 0.1 * loss_smooth
    out_ref[...] = total * jnp.ones((1, 1), jnp.float32)


def _head(nsum, ncnt, esum, ecnt, rho, wr1, br1, wr2, br2,
          ws1, bs1, ws2, bs2, wm1, bm1, wm2, bm2, tgt_s, tgt_m,
          interpret=False):
    return pl.pallas_call(
        _head_body,
        out_shape=jax.ShapeDtypeStruct((1, 1), jnp.float32),
        interpret=interpret,
    )(nsum, ncnt, esum, ecnt, rho, wr1, br1, wr2, br2,
      ws1, bs1, ws2, bs2, wm1, bm1, wm2, bm2, tgt_s, tgt_m)


def _msg_body(p1_hbm, p2_hbm, p3_hbm, src_hbm, dst_hbm,
              m_out, s_out,
              idxs, idxd, g1, g2, g3, mbuf, zbuf, acc,
              sem0, sem1, sem2, semw0, semw1):
    c = lax.axis_index("c")
    s = lax.axis_index("s")
    wid = c * NS + s

    row0 = s * 624
    nrows = jnp.where(s == NS - 1, 640, 624)
    ngrp = nrows // 16

    def zrow(r, carry):
        for j in range(D // 16):
            zbuf[r, pl.ds(j * 16, 16)] = jnp.zeros((16,), jnp.float32)
        return carry
    lax.fori_loop(0, zbuf.shape[0], zrow, 0)

    def zcp(g, carry):
        pltpu.sync_copy(zbuf, acc.at[pl.ds(row0 + g * 16, 16)])
        return carry
    lax.fori_loop(0, ngrp, zcp, 0)

    plsc.subcore_barrier()

    def chunk(t, carry):
        base = wid * EW + t * CH
        pltpu.sync_copy(src_hbm.at[pl.ds(base, CH)], idxs)
        pltpu.sync_copy(dst_hbm.at[pl.ds(base, CH)], idxd)
        cp1 = pltpu.async_copy(p1_hbm.at[idxs], g1, sem0)
        cp2 = pltpu.async_copy(p2_hbm.at[idxd], g2, sem1)
        cp3 = pltpu.async_copy(p3_hbm.at[pl.ds(base, CH)], g3, sem2)
        cp1.wait()
        cp2.wait()
        cp3.wait()

        def row(r, carry2):
            for j in range(D // 16):
                sl = pl.ds(j * 16, 16)
                x = g1[r, sl] + g2[r, sl] + g3[r, sl]
                mbuf[r, sl] = jnp.where(x > 0, x, jnp.exp(x) - 1.0)
            return carry2
        lax.fori_loop(0, CH, row, 0)

        wm = pltpu.async_copy(mbuf, m_out.at[pl.ds(base, CH)], semw0)
        wsc = pltpu.async_copy(mbuf, acc.at[idxd], semw1, add=True)
        wm.wait()
        wsc.wait()
        return carry
    lax.fori_loop(0, NCHUNK, chunk, 0)

    plsc.subcore_barrier()

    def wcp(g, carry):
        pltpu.sync_copy(acc.at[pl.ds(row0 + g * 16, 16)],
                        s_out.at[pl.ds(c * N + row0 + g * 16, 16)])
        return carry
    lax.fori_loop(0, ngrp, wcp, 0)


@functools.lru_cache(maxsize=None)
def _msg_kernel(interpret=False):
    out_type = [jax.ShapeDtypeStruct((E, D), jnp.float32),
                jax.ShapeDtypeStruct((2 * N, D), jnp.float32)]
    scratch = [pltpu.VMEM((CH,), jnp.int32),
               pltpu.VMEM((CH,), jnp.int32),
               pltpu.VMEM((CH, D), jnp.float32),
               pltpu.VMEM((CH, D), jnp.float32),
               pltpu.VMEM((CH, D), jnp.float32),
               pltpu.VMEM((CH, D), jnp.float32),
               pltpu.VMEM((16, D), jnp.float32),
               pltpu.VMEM_SHARED((N, D), jnp.float32)]
    sems = [pltpu.SemaphoreType.DMA] * 5
    mesh = plsc.VectorSubcoreMesh(core_axis_name="c", subcore_axis_name="s")
    return pl.kernel(_msg_body, out_type=out_type, mesh=mesh,
                     scratch_types=scratch + sems,
                     interpret=interpret)


def _deg_body(dst_hbm, deg_out, idxd, obuf, zbuf, dacc, semw):
    c = lax.axis_index("c")
    s = lax.axis_index("s")
    wid = c * NS + s
    row0 = s * 624
    nrows = jnp.where(s == NS - 1, 640, 624)
    ngrp = nrows // 16

    def fill(r, carry):
        for j in range(D // 16):
            zbuf[r, pl.ds(j * 16, 16)] = jnp.zeros((16,), jnp.float32)
        return carry
    lax.fori_loop(0, 16, fill, 0)

    def orow(r, carry):
        for j in range(D // 16):
            obuf[r, pl.ds(j * 16, 16)] = jnp.ones((16,), jnp.float32)
        return carry
    lax.fori_loop(0, CH, orow, 0)

    def zcp(g, carry):
        pltpu.sync_copy(zbuf, dacc.at[pl.ds(row0 + g * 16, 16)])
        return carry
    lax.fori_loop(0, ngrp, zcp, 0)

    plsc.subcore_barrier()

    def chunk(t, carry):
        base = wid * EW + t * CH
        pltpu.sync_copy(dst_hbm.at[pl.ds(base, CH)], idxd)
        pltpu.async_copy(obuf, dacc.at[idxd], semw, add=True).wait()
        return carry
    lax.fori_loop(0, NCHUNK, chunk, 0)

    plsc.subcore_barrier()

    def wcp(g, carry):
        pltpu.sync_copy(dacc.at[pl.ds(row0 + g * 16, 16)],
                        deg_out.at[pl.ds(c * N + row0 + g * 16, 16)])
        return carry
    lax.fori_loop(0, ngrp, wcp, 0)


@functools.lru_cache(maxsize=None)
def _deg_kernel(interpret=False):
    mesh = plsc.VectorSubcoreMesh(core_axis_name="c", subcore_axis_name="s")
    return pl.kernel(_deg_body,
                     out_type=[jax.ShapeDtypeStruct((2 * N, D), jnp.float32)],
                     mesh=mesh,
                     scratch_types=[pltpu.VMEM((CH,), jnp.int32),
                                    pltpu.VMEM((CH, D), jnp.float32),
                                    pltpu.VMEM((16, D), jnp.float32),
                                    pltpu.VMEM_SHARED((N, D), jnp.float32),
                                    pltpu.SemaphoreType.DMA],
                     interpret=interpret)


def kernel(feats_node, feats_edge, edge_index, graph_node_index,
           graph_edge_index, feats_rho, curve_c_shape, curve_c_magnitude,
           W_in_n, b_in_n, W_in_e, b_in_e, W_msg, b_msg, W_upd, b_upd,
           W_eupd, b_eupd, W_r1, b_r1, W_r2, b_r2, W_s1, b_s1, W_s2, b_s2,
           W_m1, b_m1, W_m2, b_m2):
    src = edge_index[0]
    dst = edge_index[1]
    row = lambda v: v.reshape(1, -1)

    h, p1, p2 = _node_embed(feats_node, W_in_n, row(b_in_n),
                            W_msg[0, :D, :], W_msg[0, D:2 * D, :])
    e, p3 = _edge_embed(feats_edge, W_in_e, row(b_in_e),
                        W_msg[0, 2 * D:, :], row(b_msg[0]))

    degs, = _deg_kernel()(dst)
    d0, d1 = degs[:N], degs[N:]
    for l in range(L):
        last = l == L - 1
        m, ssum = _msg_kernel()(p1, p2, p3, src, dst)
        w1n = W_msg[min(l + 1, L - 1), :D, :]
        w2n = W_msg[min(l + 1, L - 1), D:2 * D, :]
        h, p1, p2 = _update(h, ssum[:N], ssum[N:], d0, d1,
                            W_upd[l, :D, :], W_upd[l, D:, :], row(b_upd[l]),
                            w1n, w2n, last)
        e, p3 = _eupd(e, m, W_eupd[l, :D, :], W_eupd[l, D:, :],
                      row(b_eupd[l]), W_msg[min(l + 1, L - 1), 2 * D:, :],
                      row(b_msg[min(l + 1, L - 1)]), last)

    nsum, ncnt = _pool(graph_node_index.reshape(N // BN, 1, BN), h, BN)
    esum, ecnt = _pool(graph_edge_index.reshape(E // BE, 1, BE), e, BE)
    loss = _head(nsum, ncnt, esum, ecnt, feats_rho.reshape(B, 1),
                 W_r1, row(b_r1), W_r2, row(b_r2),
                 W_s1, row(b_s1), W_s2, row(b_s2),
                 W_m1, row(b_m1), W_m2, row(b_m2),
                 curve_c_shape, curve_c_magnitude)
    return loss[0, 0]


# speedup vs baseline: 2.8788x; 1.0460x over previous
import functools

import jax
import jax.numpy as jnp
from jax import lax
from jax.experimental import pallas as pl
from jax.experimental.pallas import tpu as pltpu
from jax.experimental.pallas import tpu_sc as plsc

N = 10000
E = 320000
B = 8
DIN = 128
DE = 16
D = 128
L = 3
R = 1024

BN = 1000
BE = 2560
NC = 2
NS = 16
NW = NC * NS
EW = E // NW
CH = 80
NCHUNK = EW // CH
NROWT = N // NS


def _elu(x):
    return jnp.where(x > 0, x, jnp.exp(x) - 1.0)


def _node_embed_body(x_ref, win_ref, bin_ref, w1_ref, w2_ref,
                     h_ref, p1_ref, p2_ref):
    h = _elu(jnp.dot(x_ref[...], win_ref[...],
                     preferred_element_type=jnp.float32) + bin_ref[...])
    h_ref[...] = h
    p1_ref[...] = jnp.dot(h, w1_ref[...], preferred_element_type=jnp.float32)
    p2_ref[...] = jnp.dot(h, w2_ref[...], preferred_element_type=jnp.float32)


def _node_embed(x, win, bin_, w1, w2, interpret=False):
    full = lambda s: pl.BlockSpec(s, lambda i: (0, 0))
    return pl.pallas_call(
        _node_embed_body,
        grid=(N // BN,),
        in_specs=[pl.BlockSpec((BN, DIN), lambda i: (i, 0)),
                  full((DIN, D)), full((1, D)), full((D, D)), full((D, D))],
        out_specs=[pl.BlockSpec((BN, D), lambda i: (i, 0))] * 3,
        out_shape=[jax.ShapeDtypeStruct((N, D), jnp.float32)] * 3,
        interpret=interpret,
    )(x, win, bin_, w1, w2)


def _edge_embed_body(x_ref, win_ref, bin_ref, w3_ref, bm_ref, e_ref, p3_ref):
    e = _elu(jnp.dot(x_ref[...], win_ref[...],
                     preferred_element_type=jnp.float32) + bin_ref[...])
    e_ref[...] = e
    p3_ref[...] = jnp.dot(e, w3_ref[...],
                          preferred_element_type=jnp.float32) + bm_ref[...]


def _edge_embed(x, win, bin_, w3, bm, interpret=False):
    full = lambda s: pl.BlockSpec(s, lambda i: (0, 0))
    return pl.pallas_call(
        _edge_embed_body,
        grid=(E // BE,),
        in_specs=[pl.BlockSpec((BE, DE), lambda i: (i, 0)),
                  full((DE, D)), full((1, D)), full((D, D)), full((1, D))],
        out_specs=[pl.BlockSpec((BE, D), lambda i: (i, 0))] * 2,
        out_shape=[jax.ShapeDtypeStruct((E, D), jnp.float32)] * 2,
        interpret=interpret,
    )(x, win, bin_, w3, bm)


def _update_body(h_ref, s0_ref, s1_ref, d0_ref, d1_ref,
                 wu1_ref, wu2_ref, bu_ref, w1_ref, w2_ref,
                 h_out, p1_out, p2_out):
    deg = jnp.maximum(d0_ref[:, :1] + d1_ref[:, :1], 1.0)
    agg = (s0_ref[...] + s1_ref[...]) / deg
    h = _elu(jnp.dot(h_ref[...], wu1_ref[...], preferred_element_type=jnp.float32)
             + jnp.dot(agg, wu2_ref[...], preferred_element_type=jnp.float32)
             + bu_ref[...])
    h_out[...] = h
    if p1_out is not None:
        p1_out[...] = jnp.dot(h, w1_ref[...], preferred_element_type=jnp.float32)
        p2_out[...] = jnp.dot(h, w2_ref[...], preferred_element_type=jnp.float32)


def _update(h, s0, s1, d0, d1, wu1, wu2, bu, w1, w2, last, interpret=False):
    full = lambda s: pl.BlockSpec(s, lambda i: (0, 0))
    nout = 1 if last else 3
    body = _update_body if not last else (
        lambda a, b, c, d, e, f, g, hh, i, j, k:
        _update_body(a, b, c, d, e, f, g, hh, i, j, k, None, None))
    out = pl.pallas_call(
        body,
        grid=(N // BN,),
        in_specs=[pl.BlockSpec((BN, D), lambda i: (i, 0))] * 3
        + [pl.BlockSpec((BN, 128), lambda i: (i, 0))] * 2
        + [full((D, D)), full((D, D)), full((1, D)), full((D, D)), full((D, D))],
        out_specs=[pl.BlockSpec((BN, D), lambda i: (i, 0))] * nout,
        out_shape=[jax.ShapeDtypeStruct((N, D), jnp.float32)] * nout,
        interpret=interpret,
    )(h, s0, s1, d0, d1, wu1, wu2, bu, w1, w2)
    return (out[0], None, None) if last else (out[0], out[1], out[2])


def _eupd_body(e_ref, m_ref, we1_ref, we2_ref, be_ref, w3_ref, bm_ref,
               e_out, p3_out):
    e = _elu(jnp.dot(e_ref[...], we1_ref[...], preferred_element_type=jnp.float32)
             + jnp.dot(m_ref[...], we2_ref[...], preferred_element_type=jnp.float32)
             + be_ref[...])
    e_out[...] = e
    if p3_out is not None:
        p3_out[...] = jnp.dot(e, w3_ref[...],
                              preferred_element_type=jnp.float32) + bm_ref[...]


def _eupd(e, m, we1, we2, be, w3, bm, last, interpret=False):
    full = lambda s: pl.BlockSpec(s, lambda i: (0, 0))
    nout = 1 if last else 2
    body = _eupd_body if not last else (
        lambda a, b, c, d, ee, f, g, hh: _eupd_body(a, b, c, d, ee, f, g, hh, None))
    out = pl.pallas_call(
        body,
        grid=(E // BE,),
        in_specs=[pl.BlockSpec((BE, D), lambda i: (i, 0))] * 2
        + [full((D, D)), full((D, D)), full((1, D)), full((D, D)), full((1, D))],
        out_specs=[pl.BlockSpec((BE, D), lambda i: (i, 0))] * nout,
        out_shape=[jax.ShapeDtypeStruct((E, D), jnp.float32)] * nout,
        interpret=interpret,
    )(e, m, we1, we2, be, w3, bm)
    return (out[0], None) if last else (out[0], out[1])


def _pool_body(gid_ref, x_ref, sum_ref, cnt_ref):
    i = pl.program_id(0)
    gid = gid_ref[0, :, :]
    rows = gid_ref.shape[2]
    onehot = (lax.broadcasted_iota(jnp.int32, (B, rows), 0)
              == gid).astype(jnp.float32)
    psum = jnp.dot(onehot, x_ref[...], preferred_element_type=jnp.float32)
    pcnt = jnp.sum(onehot, axis=1, keepdims=True)

    @pl.when(i == 0)
    def _():
        sum_ref[...] = jnp.zeros_like(sum_ref)
        cnt_ref[...] = jnp.zeros_like(cnt_ref)

    sum_ref[...] += psum
    cnt_ref[...] += jnp.broadcast_to(pcnt, cnt_ref.shape)


def _pool(gid3, x, blk, interpret=False):
    nblk = x.shape[0] // blk
    return pl.pallas_call(
        _pool_body,
        grid=(nblk,),
        in_specs=[pl.BlockSpec((1, 1, blk), lambda i: (i, 0, 0)),
                  pl.BlockSpec((blk, D), lambda i: (i, 0))],
        out_specs=[pl.BlockSpec((B, D), lambda i: (0, 0)),
                   pl.BlockSpec((B, 128), lambda i: (0, 0))],
        out_shape=[jax.ShapeDtypeStruct((B, D), jnp.float32),
                   jax.ShapeDtypeStruct((B, 128), jnp.float32)],
        interpret=interpret,
    )(gid3, x)


def _head_body(nsum_ref, ncnt_ref, esum_ref, ecnt_ref, rho_ref,
               wr1_ref, br1_ref, wr2_ref, br2_ref,
               ws1_ref, bs1_ref, ws2_ref, bs2_ref,
               wm1_ref, bm1_ref, wm2_ref, bm2_ref,
               tgt_s_ref, tgt_m_ref, out_ref):
    pn = nsum_ref[...] / jnp.maximum(ncnt_ref[:, :1], 1.0)
    pe = esum_ref[...] / jnp.maximum(ecnt_ref[:, :1], 1.0)
    emb = jnp.dot(_elu(rho_ref[...] * wr1_ref[...] + br1_ref[...]),
                  wr2_ref[...], preferred_element_type=jnp.float32) + br2_ref[...]
    x = pn + pe + emb
    cs = jnp.dot(_elu(jnp.dot(x, ws1_ref[...], preferred_element_type=jnp.float32)
                      + bs1_ref[...]),
                 ws2_ref[...], preferred_element_type=jnp.float32) + bs2_ref[...]
    cm = jnp.dot(_elu(jnp.dot(x, wm1_ref[...], preferred_element_type=jnp.float32)
                      + bm1_ref[...]),
                 wm2_ref[...], preferred_element_type=jnp.float32) + bm2_ref[...]
    loss_shape = jnp.sum((cs - tgt_s_ref[...]) ** 2) / (B * R)
    loss_mag = jnp.sum((cm - tgt_m_ref[...]) ** 2) / (B * 2)
    d = cs[:, :-1] - cs[:, 1:]
    loss_smooth = jnp.sum(d * d) / (B * (R - 1))
    total = loss_mag + loss_shape + 0.1 * loss_smooth
    out_ref[...] = total * jnp.ones((1, 1), jnp.float32)


def _head(nsum, ncnt, esum, ecnt, rho, wr1, br1, wr2, br2,
          ws1, bs1, ws2, bs2, wm1, bm1, wm2, bm2, tgt_s, tgt_m,
          interpret=False):
    return pl.pallas_call(
        _head_body,
        out_shape=jax.ShapeDtypeStruct((1, 1), jnp.float32),
        interpret=interpret,
    )(nsum, ncnt, esum, ecnt, rho, wr1, br1, wr2, br2,
      ws1, bs1, ws2, bs2, wm1, bm1, wm2, bm2, tgt_s, tgt_m)


def _msg_body(p1_hbm, p2_hbm, p3_hbm, src_hbm, dst_hbm,
              m_out, s_out,
              idxs0, idxd0, idxs1, idxd1, g1, g2, mb0, mb1, zbuf, acc,
              semg1, semg2, semg3, semwm0, semws0, semwm1, semws1):
    c = lax.axis_index("c")
    s = lax.axis_index("s")
    wid = c * NS + s
    idxs = [idxs0, idxs1]
    idxd = [idxd0, idxd1]
    mb = [mb0, mb1]
    semwm = [semwm0, semwm1]
    semws = [semws0, semws1]

    row0 = s * 624
    nrows = jnp.where(s == NS - 1, 640, 624)
    ngrp = nrows // 16

    def zrow(r, carry):
        for j in range(D // 16):
            zbuf[r, pl.ds(j * 16, 16)] = jnp.zeros((16,), jnp.float32)
        return carry
    lax.fori_loop(0, zbuf.shape[0], zrow, 0)

    def zcp(g, carry):
        pltpu.sync_copy(zbuf, acc.at[pl.ds(row0 + g * 16, 16)])
        return carry
    lax.fori_loop(0, ngrp, zcp, 0)

    plsc.subcore_barrier()

    def compute(p):
        def row(r, carry2):
            for j in range(D // 16):
                sl = pl.ds(j * 16, 16)
                x = g1[r, sl] + g2[r, sl] + mb[p][r, sl]
                mb[p][r, sl] = jnp.where(x > 0, x, jnp.exp(x) - 1.0)
            return carry2
        lax.fori_loop(0, CH, row, 0)

    WIN = 5

    def window(w, carry):
        t0 = w * WIN
        dw = None
        for i in range(WIN):
            t = t0 + i
            p = i % 2
            base = wid * EW + t * CH
            pltpu.sync_copy(src_hbm.at[pl.ds(base, CH)], idxs[p])
            pltpu.sync_copy(dst_hbm.at[pl.ds(base, CH)], idxd[p])
            dg = [pltpu.async_copy(p1_hbm.at[idxs[p]], g1, semg1),
                  pltpu.async_copy(p2_hbm.at[idxd[p]], g2, semg2),
                  pltpu.async_copy(p3_hbm.at[pl.ds(base, CH)], mb[p], semg3)]
            for d in dg:
                d.wait()
            if dw is not None:
                for d in dw:
                    d.wait()
            compute(p)
            dw = [pltpu.async_copy(mb[p], m_out.at[pl.ds(base, CH)], semwm[p]),
                  pltpu.async_copy(mb[p], acc.at[idxd[p]], semws[p], add=True)]
        for d in dw:
            d.wait()
        return carry
    lax.fori_loop(0, NCHUNK // WIN, window, 0)

    plsc.subcore_barrier()

    def wcp(g, carry):
        pltpu.sync_copy(acc.at[pl.ds(row0 + g * 16, 16)],
                        s_out.at[pl.ds(c * N + row0 + g * 16, 16)])
        return carry
    lax.fori_loop(0, ngrp, wcp, 0)


@functools.lru_cache(maxsize=None)
def _msg_kernel(interpret=False):
    out_type = [jax.ShapeDtypeStruct((E, D), jnp.float32),
                jax.ShapeDtypeStruct((2 * N, D), jnp.float32)]
    scratch = ([pltpu.VMEM((CH,), jnp.int32)] * 4
               + [pltpu.VMEM((CH, D), jnp.float32)] * 4
               + [pltpu.VMEM((16, D), jnp.float32),
                  pltpu.VMEM_SHARED((N, D), jnp.float32)])
    sems = [pltpu.SemaphoreType.DMA] * 7
    mesh = plsc.VectorSubcoreMesh(core_axis_name="c", subcore_axis_name="s")
    return pl.kernel(_msg_body, out_type=out_type, mesh=mesh,
                     scratch_types=scratch + sems,
                     interpret=interpret)


def _deg_body(dst_hbm, deg_out, idxd, obuf, zbuf, dacc, semw):
    c = lax.axis_index("c")
    s = lax.axis_index("s")
    wid = c * NS + s
    row0 = s * 624
    nrows = jnp.where(s == NS - 1, 640, 624)
    ngrp = nrows // 16

    def fill(r, carry):
        for j in range(D // 16):
            zbuf[r, pl.ds(j * 16, 16)] = jnp.zeros((16,), jnp.float32)
        return carry
    lax.fori_loop(0, 16, fill, 0)

    def orow(r, carry):
        for j in range(D // 16):
            obuf[r, pl.ds(j * 16, 16)] = jnp.ones((16,), jnp.float32)
        return carry
    lax.fori_loop(0, CH, orow, 0)

    def zcp(g, carry):
        pltpu.sync_copy(zbuf, dacc.at[pl.ds(row0 + g * 16, 16)])
        return carry
    lax.fori_loop(0, ngrp, zcp, 0)

    plsc.subcore_barrier()

    def chunk(t, carry):
        base = wid * EW + t * CH
        pltpu.sync_copy(dst_hbm.at[pl.ds(base, CH)], idxd)
        pltpu.async_copy(obuf, dacc.at[idxd], semw, add=True).wait()
        return carry
    lax.fori_loop(0, NCHUNK, chunk, 0)

    plsc.subcore_barrier()

    def wcp(g, carry):
        pltpu.sync_copy(dacc.at[pl.ds(row0 + g * 16, 16)],
                        deg_out.at[pl.ds(c * N + row0 + g * 16, 16)])
        return carry
    lax.fori_loop(0, ngrp, wcp, 0)


@functools.lru_cache(maxsize=None)
def _deg_kernel(interpret=False):
    mesh = plsc.VectorSubcoreMesh(core_axis_name="c", subcore_axis_name="s")
    return pl.kernel(_deg_body,
                     out_type=[jax.ShapeDtypeStruct((2 * N, D), jnp.float32)],
                     mesh=mesh,
                     scratch_types=[pltpu.VMEM((CH,), jnp.int32),
                                    pltpu.VMEM((CH, D), jnp.float32),
                                    pltpu.VMEM((16, D), jnp.float32),
                                    pltpu.VMEM_SHARED((N, D), jnp.float32),
                                    pltpu.SemaphoreType.DMA],
                     interpret=interpret)


def kernel(feats_node, feats_edge, edge_index, graph_node_index,
           graph_edge_index, feats_rho, curve_c_shape, curve_c_magnitude,
           W_in_n, b_in_n, W_in_e, b_in_e, W_msg, b_msg, W_upd, b_upd,
           W_eupd, b_eupd, W_r1, b_r1, W_r2, b_r2, W_s1, b_s1, W_s2, b_s2,
           W_m1, b_m1, W_m2, b_m2):
    src = edge_index[0]
    dst = edge_index[1]
    row = lambda v: v.reshape(1, -1)

    h, p1, p2 = _node_embed(feats_node, W_in_n, row(b_in_n),
                            W_msg[0, :D, :], W_msg[0, D:2 * D, :])
    e, p3 = _edge_embed(feats_edge, W_in_e, row(b_in_e),
                        W_msg[0, 2 * D:, :], row(b_msg[0]))

    degs, = _deg_kernel()(dst)
    d0, d1 = degs[:N], degs[N:]
    for l in range(L):
        last = l == L - 1
        m, ssum = _msg_kernel()(p1, p2, p3, src, dst)
        w1n = W_msg[min(l + 1, L - 1), :D, :]
        w2n = W_msg[min(l + 1, L - 1), D:2 * D, :]
        h, p1, p2 = _update(h, ssum[:N], ssum[N:], d0, d1,
                            W_upd[l, :D, :], W_upd[l, D:, :], row(b_upd[l]),
                            w1n, w2n, last)
        e, p3 = _eupd(e, m, W_eupd[l, :D, :], W_eupd[l, D:, :],
                      row(b_eupd[l]), W_msg[min(l + 1, L - 1), 2 * D:, :],
                      row(b_msg[min(l + 1, L - 1)]), last)

    nsum, ncnt = _pool(graph_node_index.reshape(N // BN, 1, BN), h, BN)
    esum, ecnt = _pool(graph_edge_index.reshape(E // BE, 1, BE), e, BE)
    loss = _head(nsum, ncnt, esum, ecnt, feats_rho.reshape(B, 1),
                 W_r1, row(b_r1), W_r2, row(b_r2),
                 W_s1, row(b_s1), W_s2, row(b_s2),
                 W_m1, row(b_m1), W_m2, row(b_m2),
                 curve_c_shape, curve_c_magnitude)
    return loss[0, 0]
